# baseline (speedup 1.0000x reference)
import functools

import jax
import jax.numpy as jnp
from jax import lax
from jax.experimental import pallas as pl
from jax.experimental.pallas import tpu as pltpu
from jax.experimental.pallas import tpu_sc as plsc

NN = 2048
EE = 65536
DF = 128

NC = 2
NS = 16
NW = NC * NS
LL = 16
ROWS = 32
PASSES = NN // (NW * ROWS)
ECHUNK = 16384

_SC_MESH = plsc.VectorSubcoreMesh(core_axis_name="c", subcore_axis_name="s")

EPT = EE // NS
NPASS = 8
QROWS = NN // (NC * NPASS)
QW = QROWS * NN
QPAD = 128
SLICE = QW // NS


@functools.partial(
    pl.kernel,
    out_type=jax.ShapeDtypeStruct((NN * NN,), jnp.float32),
    mesh=_SC_MESH,
    scratch_types=[
        pltpu.VMEM((SLICE,), jnp.float32),
        pltpu.VMEM((EPT,), jnp.int32),
        pltpu.VMEM((EPT,), jnp.int32),
        pltpu.VMEM((EPT,), jnp.int32),
        pltpu.VMEM((EPT,), jnp.float32),
        pltpu.VMEM_SHARED((QW + QPAD,), jnp.float32),
        pltpu.VMEM_SHARED((QW + QPAD,), jnp.float32),
        pltpu.SemaphoreType.DMA,
        pltpu.SemaphoreType.DMA,
        pltpu.SemaphoreType.DMA,
    ],
    compiler_params=pltpu.CompilerParams(needs_layout_passes=False),
)
def _build_mask(src_hbm, dst_hbm, mask_hbm, zbuf, srcv, dstv, idx2, ones_v,
                spa, spb, ssem, wsa, wsb):
    cid = lax.axis_index("c")
    sid = lax.axis_index("s")
    zeros16 = jnp.zeros((LL,), jnp.float32)
    ones16 = jnp.ones((LL,), jnp.float32)

    off = sid * EPT
    pltpu.sync_copy(src_hbm.at[pl.ds(off, EPT)], srcv)
    pltpu.sync_copy(dst_hbm.at[pl.ds(off, EPT)], dstv)

    def ob(t, carry):
        ones_v[pl.ds(t * LL, LL)] = ones16
        return carry

    lax.fori_loop(0, EPT // LL, ob, 0, unroll=8)

    def zb(t, carry):
        zbuf[pl.ds(t * LL, LL)] = zeros16
        return carry

    lax.fori_loop(0, SLICE // LL, zb, 0, unroll=8)
    sbase = pl.multiple_of(sid * SLICE, 8)
    garb16 = QW + lax.iota(jnp.int32, LL)

    wb = [None, None]
    for q in range(NPASS):
        spq = spa if q % 2 == 0 else spb
        wsem = wsa if q % 2 == 0 else wsb
        if wb[q % 2] is not None:
            wb[q % 2].wait()
        pltpu.sync_copy(zbuf, spq.at[pl.ds(sbase, SLICE)])
        plsc.subcore_barrier()

        base = (cid * NPASS + q) * QW

        def ib(t, carry):
            s16 = srcv[pl.ds(t * LL, LL)]
            d16 = dstv[pl.ds(t * LL, LL)]
            rel = (s16 * NN + d16) - base
            m = (rel >= 0) & (rel < QW)
            idx2[pl.ds(t * LL, LL)] = jnp.where(m, rel, garb16)
            return carry

        lax.fori_loop(0, EPT // LL, ib, 0, unroll=8)

        pltpu.async_copy(ones_v, spq.at[idx2], ssem).wait()
        plsc.subcore_barrier()

        wb[q % 2] = pltpu.async_copy(
            spq.at[pl.ds(sbase, SLICE)],
            mask_hbm.at[pl.ds(pl.multiple_of(base + sbase, 8), SLICE)],
            wsem,
        )
    wb[0].wait()
    wb[1].wait()


BI2 = 256
GI2 = NN // BI2
BJ = 512
GJ = NN // BJ
_DN0 = (((0,), (0,)), ((), ()))


def _powmask_body(m1_ref, mask2_ref, cs1_ref, cs2_ref, mbf_ref):
    i = pl.program_id(0)

    @pl.when(i == 0)
    def _():
        mbf_ref[...] = m1_ref[...].astype(jnp.bfloat16)
        cs1_ref[...] = jax.lax.dot_general(
            m1_ref[...], jnp.ones((NN, 1), jnp.float32), _DN0,
            preferred_element_type=jnp.float32,
        )

    lhs = mbf_ref[pl.ds(i * BI2, BI2), :]
    c = jax.lax.dot(lhs, mbf_ref[...], preferred_element_type=jnp.float32)
    m2f = (c > 0.0).astype(jnp.float32)
    mask2_ref[...] = m2f.astype(jnp.bfloat16)
    part = jax.lax.dot_general(
        m2f, jnp.ones((BI2, 1), jnp.float32), _DN0,
        preferred_element_type=jnp.float32,
    )

    @pl.when(i == 0)
    def _():
        cs2_ref[...] = part

    @pl.when(i != 0)
    def _():
        cs2_ref[...] += part


_powmask = pl.pallas_call(
    _powmask_body,
    grid=(GI2,),
    in_specs=[
        pl.BlockSpec((NN, NN), lambda i: (0, 0)),
    ],
    out_specs=[
        pl.BlockSpec((BI2, NN), lambda i: (i, 0)),
        pl.BlockSpec((NN, 1), lambda i: (0, 0)),
        pl.BlockSpec((NN, 1), lambda i: (0, 0)),
    ],
    out_shape=[
        jax.ShapeDtypeStruct((NN, NN), jnp.bfloat16),
        jax.ShapeDtypeStruct((NN, 1), jnp.float32),
        jax.ShapeDtypeStruct((NN, 1), jnp.float32),
    ],
    scratch_shapes=[pltpu.VMEM((NN, NN), jnp.bfloat16)],
)


def _agg_body(m1_ref, m2_ref, x_ref, w_ref, b_ref, alpha_ref, cs1_ref,
              cs2_ref, out_ref):
    j = pl.program_id(0)
    a0 = alpha_ref[0]
    a1 = alpha_ref[1]
    h = jnp.dot(x_ref[...], w_ref[...], preferred_element_type=jnp.float32)
    d1 = jax.lax.rsqrt(cs1_ref[...] + 1.0)
    d2 = jax.lax.rsqrt(cs2_ref[...] + 1.0)
    g1 = (a0 * d1) * h
    g2 = (a1 * d2) * h
    s1 = jax.lax.dot_general(m1_ref[...], g1, _DN0,
                             preferred_element_type=jnp.float32)
    s2 = jax.lax.dot_general(m2_ref[...].astype(jnp.float32), g2, _DN0,
                             preferred_element_type=jnp.float32)
    d1j = jax.lax.rsqrt(cs1_ref[pl.ds(j * BJ, BJ), :] + 1.0)
    d2j = jax.lax.rsqrt(cs2_ref[pl.ds(j * BJ, BJ), :] + 1.0)
    hj = jnp.dot(x_ref[pl.ds(j * BJ, BJ), :], w_ref[...],
                 preferred_element_type=jnp.float32)
    out_ref[...] = (d1j * (s1 + (a0 * d1j) * hj)
                    + d2j * (s2 + (a1 * d2j) * hj)
                    + (a0 + a1) * b_ref[...])


_agg = pl.pallas_call(
    _agg_body,
    grid=(GJ,),
    in_specs=[
        pl.BlockSpec((NN, BJ), lambda j: (0, j)),
        pl.BlockSpec((NN, BJ), lambda j: (0, j)),
        pl.BlockSpec((NN, DF), lambda j: (0, 0)),
        pl.BlockSpec((DF, DF), lambda j: (0, 0)),
        pl.BlockSpec((1, DF), lambda j: (0, 0)),
        pl.BlockSpec(memory_space=pltpu.SMEM),
        pl.BlockSpec((NN, 1), lambda j: (0, 0)),
        pl.BlockSpec((NN, 1), lambda j: (0, 0)),
    ],
    out_specs=pl.BlockSpec((BJ, DF), lambda j: (j, 0)),
    out_shape=jax.ShapeDtypeStruct((NN, DF), jnp.float32),
)


def kernel(x, edge_index, W, b, alpha):
    src = edge_index[0]
    dst = edge_index[1]
    mask1 = _build_mask(src, dst).reshape(NN, NN)
    mask2, cs1, cs2 = _powmask(mask1)
    return _agg(mask1, mask2, x, W, b.reshape(1, DF), alpha, cs1, cs2)


# speedup vs baseline: 1.3294x; 1.3294x over previous
import functools

import jax
import jax.numpy as jnp
from jax import lax
from jax.experimental import pallas as pl
from jax.experimental.pallas import tpu as pltpu
from jax.experimental.pallas import tpu_sc as plsc

NN = 2048
EE = 65536
DF = 128

NC = 2
NS = 16
NW = NC * NS
LL = 16
ROWS = 32
PASSES = NN // (NW * ROWS)
ECHUNK = 16384

_SC_MESH = plsc.VectorSubcoreMesh(core_axis_name="c", subcore_axis_name="s")

EPT = EE // NS
R0 = 440
QW0 = R0 * NN
QW2 = (NN // NC - 2 * R0) * NN
QPAD = 16
ZW = QW0 // NS


@functools.partial(
    pl.kernel,
    out_type=jax.ShapeDtypeStruct((NN * NN,), jnp.float32),
    mesh=_SC_MESH,
    scratch_types=[
        pltpu.VMEM((ZW,), jnp.float32),
        pltpu.VMEM((EPT,), jnp.int32),
        pltpu.VMEM((EPT,), jnp.int32),
        pltpu.VMEM((EPT,), jnp.int32),
        pltpu.VMEM((EPT,), jnp.float32),
        pltpu.VMEM_SHARED((QW0 + QPAD,), jnp.float32),
        pltpu.SemaphoreType.DMA,
    ],
    compiler_params=pltpu.CompilerParams(needs_layout_passes=False),
)
def _build_mask(src_hbm, dst_hbm, mask_hbm, zbuf, srcv, dstv, idx2, ones_v,
                spq, ssem):
    cid = lax.axis_index("c")
    sid = lax.axis_index("s")
    zeros16 = jnp.zeros((LL,), jnp.float32)
    ones16 = jnp.ones((LL,), jnp.float32)

    off = sid * EPT
    pltpu.sync_copy(src_hbm.at[pl.ds(off, EPT)], srcv)
    pltpu.sync_copy(dst_hbm.at[pl.ds(off, EPT)], dstv)

    def ob(t, carry):
        ones_v[pl.ds(t * LL, LL)] = ones16
        return carry

    lax.fori_loop(0, EPT // LL, ob, 0, unroll=8)

    def zb(t, carry):
        zbuf[pl.ds(t * LL, LL)] = zeros16
        return carry

    lax.fori_loop(0, ZW // LL, zb, 0, unroll=8)

    for roff, qw in ((0, QW0), (R0, QW0), (2 * R0, QW2)):
        slice_q = qw // NS
        sbase = pl.multiple_of(sid * slice_q, 8)
        base = (cid * (NN // NC) + roff) * NN
        garb16 = qw + lax.iota(jnp.int32, LL)

        pltpu.sync_copy(zbuf.at[pl.ds(0, slice_q)],
                        spq.at[pl.ds(sbase, slice_q)])
        plsc.subcore_barrier()

        def ib(t, carry):
            s16 = srcv[pl.ds(t * LL, LL)]
            d16 = dstv[pl.ds(t * LL, LL)]
            rel = (s16 * NN + d16) - base
            m = (rel >= 0) & (rel < qw)
            idx2[pl.ds(t * LL, LL)] = jnp.where(m, rel, garb16)
            return carry

        lax.fori_loop(0, EPT // LL, ib, 0, unroll=8)

        pltpu.async_copy(ones_v, spq.at[idx2], ssem).wait()
        plsc.subcore_barrier()

        pltpu.sync_copy(
            spq.at[pl.ds(sbase, slice_q)],
            mask_hbm.at[pl.ds(pl.multiple_of(base + sbase, 8), slice_q)],
        )


BI2 = 256
GI2 = NN // BI2
BJ = 512
GJ = NN // BJ
_DN0 = (((0,), (0,)), ((), ()))


def _powmask_body(m1_ref, mask2_ref, cs1_ref, cs2_ref, mbf_ref):
    i = pl.program_id(0)

    @pl.when(i == 0)
    def _():
        mbf_ref[...] = m1_ref[...].astype(jnp.bfloat16)
        cs1_ref[...] = jax.lax.dot_general(
            m1_ref[...], jnp.ones((NN, 1), jnp.float32), _DN0,
            preferred_element_type=jnp.float32,
        )

    lhs = mbf_ref[pl.ds(i * BI2, BI2), :]
    c = jax.lax.dot(lhs, mbf_ref[...], preferred_element_type=jnp.float32)
    m2f = (c > 0.0).astype(jnp.float32)
    mask2_ref[...] = m2f.astype(jnp.bfloat16)
    part = jax.lax.dot_general(
        m2f, jnp.ones((BI2, 1), jnp.float32), _DN0,
        preferred_element_type=jnp.float32,
    )

    @pl.when(i == 0)
    def _():
        cs2_ref[...] = part

    @pl.when(i != 0)
    def _():
        cs2_ref[...] += part


_powmask = pl.pallas_call(
    _powmask_body,
    grid=(GI2,),
    in_specs=[
        pl.BlockSpec((NN, NN), lambda i: (0, 0)),
    ],
    out_specs=[
        pl.BlockSpec((BI2, NN), lambda i: (i, 0)),
        pl.BlockSpec((NN, 1), lambda i: (0, 0)),
        pl.BlockSpec((NN, 1), lambda i: (0, 0)),
    ],
    out_shape=[
        jax.ShapeDtypeStruct((NN, NN), jnp.bfloat16),
        jax.ShapeDtypeStruct((NN, 1), jnp.float32),
        jax.ShapeDtypeStruct((NN, 1), jnp.float32),
    ],
    scratch_shapes=[pltpu.VMEM((NN, NN), jnp.bfloat16)],
)


def _agg_body(m1_ref, m2_ref, x_ref, w_ref, b_ref, alpha_ref, cs1_ref,
              cs2_ref, out_ref):
    j = pl.program_id(0)
    a0 = alpha_ref[0]
    a1 = alpha_ref[1]
    h = jnp.dot(x_ref[...], w_ref[...], preferred_element_type=jnp.float32)
    d1 = jax.lax.rsqrt(cs1_ref[...] + 1.0)
    d2 = jax.lax.rsqrt(cs2_ref[...] + 1.0)
    g1 = (a0 * d1) * h
    g2 = (a1 * d2) * h
    s1 = jax.lax.dot_general(m1_ref[...], g1, _DN0,
                             preferred_element_type=jnp.float32)
    s2 = jax.lax.dot_general(m2_ref[...].astype(jnp.float32), g2, _DN0,
                             preferred_element_type=jnp.float32)
    d1j = jax.lax.rsqrt(cs1_ref[pl.ds(j * BJ, BJ), :] + 1.0)
    d2j = jax.lax.rsqrt(cs2_ref[pl.ds(j * BJ, BJ), :] + 1.0)
    hj = jnp.dot(x_ref[pl.ds(j * BJ, BJ), :], w_ref[...],
                 preferred_element_type=jnp.float32)
    out_ref[...] = (d1j * (s1 + (a0 * d1j) * hj)
                    + d2j * (s2 + (a1 * d2j) * hj)
                    + (a0 + a1) * b_ref[...])


_agg = pl.pallas_call(
    _agg_body,
    grid=(GJ,),
    in_specs=[
        pl.BlockSpec((NN, BJ), lambda j: (0, j)),
        pl.BlockSpec((NN, BJ), lambda j: (0, j)),
        pl.BlockSpec((NN, DF), lambda j: (0, 0)),
        pl.BlockSpec((DF, DF), lambda j: (0, 0)),
        pl.BlockSpec((1, DF), lambda j: (0, 0)),
        pl.BlockSpec(memory_space=pltpu.SMEM),
        pl.BlockSpec((NN, 1), lambda j: (0, 0)),
        pl.BlockSpec((NN, 1), lambda j: (0, 0)),
    ],
    out_specs=pl.BlockSpec((BJ, DF), lambda j: (j, 0)),
    out_shape=jax.ShapeDtypeStruct((NN, DF), jnp.float32),
)


def kernel(x, edge_index, W, b, alpha):
    src = edge_index[0]
    dst = edge_index[1]
    mask1 = _build_mask(src, dst).reshape(NN, NN)
    mask2, cs1, cs2 = _powmask(mask1)
    return _agg(mask1, mask2, x, W, b.reshape(1, DF), alpha, cs1, cs2)
